# baseline (speedup 1.0000x reference)
import functools

import jax
import jax.numpy as jnp
from jax import lax
from jax.experimental import pallas as pl
from jax.experimental.pallas import tpu as pltpu
from jax.experimental.pallas import tpu_sc as plsc

NFFT = 65536
NP = 8192
NSRCH = NP + 1
TBASE = 8
TPAD = 8208
NC = 2
NS = 16
NW = NC * NS
CHUNK = NFFT // NW
PCHUNK = NP // NS
L = 16


def _c16(v, dtype):
    return jnp.full((L,), v, dtype)


def _sc_body(y_hbm, xp_hbm, pp_hbm, ew_hbm, abg_hbm, out_hbm,
             xp_c, pp_c, ew_c, yg_c, plc_v, hc_v, pph_v, abg_v,
             ptab_v, htab_v, out_v, plarr_s, harr_s, sem):
    cid = lax.axis_index("c")
    sid = lax.axis_index("s")
    wid = sid * NC + cid
    pbase = sid * PCHUNK

    cps = [pltpu.async_copy(pp_hbm.at[pl.ds(pbase, PCHUNK)], pp_c, sem),
           pltpu.async_copy(xp_hbm.at[pl.ds(pbase, PCHUNK)], xp_c, sem),
           pltpu.async_copy(ew_hbm.at[pl.ds(pbase, PCHUNK)], ew_c, sem),
           pltpu.async_copy(pp_hbm.at[pl.ds(0, L)], pph_v, sem),
           pltpu.async_copy(abg_hbm, abg_v, sem)]
    for cp in cps:
        cp.wait()

    iota = lax.iota(jnp.int32, L)

    def bc(ref, i):
        return plsc.load_gather(ref, [_c16(i, jnp.int32)])

    p0 = bc(pph_v, 0)
    s_shift = jnp.where(p0 == _c16(0, jnp.int32),
                        _c16(0, jnp.int32), _c16(1, jnp.int32))

    gcps = [pltpu.async_copy(y_hbm.at[pp_c.at[pl.ds(k * 128, 128)]],
                             yg_c.at[pl.ds(k * 128, 128)], sem)
            for k in range(PCHUNK // 128)]
    for cp in gcps:
        cp.wait()

    PU = 8

    def pilot_body(k, carry):
        gi_l = [(k * PU + u) * L + iota for u in range(PU)]
        pp_l = [plsc.load_gather(pp_c, [gi]) for gi in gi_l]
        xp_l = [plsc.load_gather(xp_c, [gi]) for gi in gi_l]
        ew_l = [plsc.load_gather(ew_c, [gi]) for gi in gi_l]
        yg_l = [plsc.load_gather(yg_c, [gi]) for gi in gi_l]
        for u in range(PU):
            h = yg_l[u] / xp_l[u] * ew_l[u]
            plsc.store_scatter(hc_v, [gi_l[u]], h)
            plsc.store_scatter(plc_v, [gi_l[u]],
                               (pp_l[u] + s_shift).astype(jnp.float32))
        return carry

    lax.fori_loop(0, PCHUNK // L // PU, pilot_body, 0)

    pltpu.sync_copy(plc_v, plarr_s.at[pl.ds(pbase, PCHUNK)])
    pltpu.sync_copy(hc_v, harr_s.at[pl.ds(pbase, PCHUNK)])
    plsc.subcore_barrier()
    pltpu.sync_copy(plarr_s, ptab_v.at[pl.ds(TBASE, NP)])
    pltpu.sync_copy(harr_s, htab_v.at[pl.ds(TBASE, NP)])

    pl0 = bc(ptab_v, TBASE)
    pl1v = bc(ptab_v, TBASE + 1)
    plm1 = bc(ptab_v, TBASE + NP - 1)
    plm2 = bc(ptab_v, TBASE + NP - 2)
    H0 = bc(htab_v, TBASE)
    H1 = bc(htab_v, TBASE + 1)
    Hm1 = bc(htab_v, TBASE + NP - 1)
    Hm2 = bc(htab_v, TBASE + NP - 2)
    slope0 = (H1 - H0) / (pl1v - pl0)
    first_H = jnp.where(pl0 > _c16(0.0, jnp.float32), H0 - slope0 * pl0, H0)
    slope1 = (Hm1 - Hm2) / (plm1 - plm2)
    tail_H = Hm1 + slope1 * (_c16(float(NFFT - 1), jnp.float32) - plm1)
    cond_tail = plm1 < _c16(float(NFFT - 1), jnp.float32)
    o_vec = jnp.where(cond_tail, _c16(1, jnp.int32), _c16(0, jnp.int32))
    last_p = jnp.maximum(plm1, _c16(float(NFFT - 1), jnp.float32))

    m0 = iota == _c16(0, jnp.int32)
    m01 = iota < _c16(2, jnp.int32)
    head_i = iota + _c16(TBASE - 2, jnp.int32)
    plsc.store_scatter(
        ptab_v, [head_i],
        jnp.where(m0, _c16(-1.0, jnp.float32), _c16(0.0, jnp.float32)),
        mask=m01)
    plsc.store_scatter(
        htab_v, [head_i],
        jnp.where(m0, _c16(0.0, jnp.float32), first_H),
        mask=m01)
    tail_i = _c16(TBASE + NP, jnp.int32)
    plsc.store_scatter(ptab_v, [tail_i], last_p, mask=m0)
    plsc.store_scatter(htab_v, [tail_i], tail_H, mask=m0)

    av = abg_v[pl.ds(0, L)]
    bv = abg_v[pl.ds(L, L)]
    gv = abg_v[pl.ds(2 * L, L)]
    base_out = wid * CHUNK
    g0_hi = o_vec + _c16(NP, jnp.int32)

    def count_le(idxf):
        pos = _c16(0, jnp.int32)
        bit = 1 << 13
        while bit:
            cand = pos + bit
            gidx = jnp.minimum(cand, _c16(NSRCH, jnp.int32)) \
                + _c16(TBASE - 1, jnp.int32)
            tv = plsc.load_gather(ptab_v, [gidx])
            take = jnp.logical_and(cand <= _c16(NSRCH, jnp.int32),
                                   tv <= idxf)
            pos = jnp.where(take, cand, pos)
            bit >>= 1
        return pos

    wlo = count_le((_c16(base_out, jnp.int32) - _c16(1, jnp.int32))
                   .astype(jnp.float32))
    whi = count_le(_c16(float(CHUNK - 1), jnp.float32)
                   + _c16(base_out, jnp.int32).astype(jnp.float32))
    wvec = whi - wlo
    w_s = lax.reduce_max(wvec, (0,))
    nsteps = jnp.int32(0)
    for k in range(14):
        nsteps = nsteps + jnp.where(w_s >= (1 << k), 1, 0).astype(jnp.int32)
    gbase = wlo + _c16(TBASE - 1, jnp.int32)

    OU = 8

    def out_body(i, carry):
        li_l, idxf_l, pos_l = [], [], []
        for u in range(OU):
            li = (i * OU + u) * L + iota
            li_l.append(li)
            idxf_l.append((base_out + li).astype(jnp.float32))
            pos_l.append(_c16(0, jnp.int32))

        def step(t, poss):
            bit = jnp.broadcast_to(
                lax.shift_left(jnp.int32(1), nsteps - 1 - t).astype(jnp.int32), (L,))
            poss = list(poss)
            for u in range(OU):
                cand = poss[u] + bit
                gidx = jnp.minimum(cand, wvec) + gbase
                tv = plsc.load_gather(ptab_v, [gidx])
                take = jnp.logical_and(cand <= wvec, tv <= idxf_l[u])
                poss[u] = jnp.where(take, cand, poss[u])
            return tuple(poss)

        pos_l = list(lax.fori_loop(0, nsteps, step, tuple(pos_l)))
        g0_l = [jnp.minimum(jnp.maximum(wlo + pos_l[u] + 1, o_vec), g0_hi)
                + _c16(TBASE - 2, jnp.int32) for u in range(OU)]
        x0_l = [plsc.load_gather(ptab_v, [g0_l[u]]) for u in range(OU)]
        x1_l = [plsc.load_gather(ptab_v, [g0_l[u] + 1]) for u in range(OU)]
        yb_l = [plsc.load_gather(htab_v, [g0_l[u]]) for u in range(OU)]
        ya_l = [plsc.load_gather(htab_v, [g0_l[u] + 1]) for u in range(OU)]
        for u in range(OU):
            denom = x1_l[u] - x0_l[u]
            safe = denom > _c16(0.0, jnp.float32)
            df = jnp.where(
                safe,
                (idxf_l[u] - x0_l[u])
                / jnp.where(safe, denom, _c16(1.0, jnp.float32)),
                _c16(0.0, jnp.float32))
            outv = av * ya_l[u] + bv * yb_l[u] + gv * df
            plsc.store_scatter(out_v, [li_l[u]], outv)
        return carry

    lax.fori_loop(0, CHUNK // L // OU, out_body, 0)

    pltpu.sync_copy(out_v, out_hbm.at[pl.ds(base_out, CHUNK)])


_estimator = functools.partial(
    pl.kernel,
    out_type=jax.ShapeDtypeStruct((NFFT,), jnp.float32),
    mesh=plsc.VectorSubcoreMesh(core_axis_name="c", subcore_axis_name="s",
                                num_cores=NC, num_subcores=NS),
    compiler_params=pltpu.CompilerParams(needs_layout_passes=False),
    scratch_types=[
        pltpu.VMEM((PCHUNK,), jnp.float32),
        pltpu.VMEM((PCHUNK,), jnp.int32),
        pltpu.VMEM((PCHUNK,), jnp.float32),
        pltpu.VMEM((PCHUNK,), jnp.float32),
        pltpu.VMEM((PCHUNK,), jnp.float32),
        pltpu.VMEM((PCHUNK,), jnp.float32),
        pltpu.VMEM((L,), jnp.int32),
        pltpu.VMEM((3 * L,), jnp.float32),
        pltpu.VMEM((TPAD,), jnp.float32),
        pltpu.VMEM((TPAD,), jnp.float32),
        pltpu.VMEM((CHUNK,), jnp.float32),
        pltpu.VMEM_SHARED((NP,), jnp.float32),
        pltpu.VMEM_SHARED((NP,), jnp.float32),
        pltpu.SemaphoreType.DMA,
    ],
)(_sc_body)


def kernel(Y, Xp, pilot_pos, Nfft, estimation_weights, alpha, beta, gamma):
    del Nfft
    a = jnp.broadcast_to(jnp.reshape(jnp.asarray(alpha, jnp.float32), (1,)), (L,))
    b = jnp.broadcast_to(jnp.reshape(jnp.asarray(beta, jnp.float32), (1,)), (L,))
    g = jnp.broadcast_to(jnp.reshape(jnp.asarray(gamma, jnp.float32), (1,)), (L,))
    abg = jnp.concatenate([a, b, g])
    return _estimator(Y.astype(jnp.float32), Xp.astype(jnp.float32),
                      pilot_pos.astype(jnp.int32),
                      estimation_weights.astype(jnp.float32), abg)


# speedup vs baseline: 1.0268x; 1.0268x over previous
import functools

import jax
import jax.numpy as jnp
from jax import lax
from jax.experimental import pallas as pl
from jax.experimental.pallas import tpu as pltpu
from jax.experimental.pallas import tpu_sc as plsc

NFFT = 65536
NP = 8192
NSRCH = NP + 1
TBASE = 8
TPAD = 8208
NC = 2
NS = 16
NW = NC * NS
CHUNK = NFFT // NW
PCHUNK = NP // NS
L = 16


def _c16(v, dtype):
    return jnp.full((L,), v, dtype)


def _sc_body(y_hbm, xp_hbm, pp_hbm, ew_hbm, abg_hbm, out_hbm,
             xp_c, pp_c, ew_c, yg_c, plc_v, hc_v, pph_v, abg_v,
             ptab_v, htab_v, out_v, plarr_s, harr_s, sem, sem2):
    cid = lax.axis_index("c")
    sid = lax.axis_index("s")
    wid = sid * NC + cid
    pbase = sid * PCHUNK

    cp_pp = pltpu.async_copy(pp_hbm.at[pl.ds(pbase, PCHUNK)], pp_c, sem2)
    cps = [pltpu.async_copy(xp_hbm.at[pl.ds(pbase, PCHUNK)], xp_c, sem),
           pltpu.async_copy(ew_hbm.at[pl.ds(pbase, PCHUNK)], ew_c, sem),
           pltpu.async_copy(pp_hbm.at[pl.ds(0, L)], pph_v, sem),
           pltpu.async_copy(abg_hbm, abg_v, sem)]
    cp_pp.wait()

    gcps = [pltpu.async_copy(y_hbm.at[pp_c.at[pl.ds(k * 128, 128)]],
                             yg_c.at[pl.ds(k * 128, 128)], sem2)
            for k in range(PCHUNK // 128)]
    for cp in cps:
        cp.wait()

    iota = lax.iota(jnp.int32, L)

    def bc(ref, i):
        return plsc.load_gather(ref, [_c16(i, jnp.int32)])

    p0 = bc(pph_v, 0)
    s_shift = jnp.where(p0 == _c16(0, jnp.int32),
                        _c16(0, jnp.int32), _c16(1, jnp.int32))

    for cp in gcps:
        cp.wait()

    PU = 8

    def pilot_body(k, carry):
        gi_l = [(k * PU + u) * L + iota for u in range(PU)]
        pp_l = [plsc.load_gather(pp_c, [gi]) for gi in gi_l]
        xp_l = [plsc.load_gather(xp_c, [gi]) for gi in gi_l]
        ew_l = [plsc.load_gather(ew_c, [gi]) for gi in gi_l]
        yg_l = [plsc.load_gather(yg_c, [gi]) for gi in gi_l]
        for u in range(PU):
            h = yg_l[u] / xp_l[u] * ew_l[u]
            plsc.store_scatter(hc_v, [gi_l[u]], h)
            plsc.store_scatter(plc_v, [gi_l[u]],
                               (pp_l[u] + s_shift).astype(jnp.float32))
        return carry

    lax.fori_loop(0, PCHUNK // L // PU, pilot_body, 0)

    pub = [pltpu.async_copy(plc_v, plarr_s.at[pl.ds(pbase, PCHUNK)], sem2),
           pltpu.async_copy(hc_v, harr_s.at[pl.ds(pbase, PCHUNK)], sem2)]
    for cp in pub:
        cp.wait()
    plsc.subcore_barrier()
    pulls = [pltpu.async_copy(plarr_s, ptab_v.at[pl.ds(TBASE, NP)], sem2),
             pltpu.async_copy(harr_s, htab_v.at[pl.ds(TBASE, NP)], sem2)]
    for cp in pulls:
        cp.wait()

    pl0 = bc(ptab_v, TBASE)
    pl1v = bc(ptab_v, TBASE + 1)
    plm1 = bc(ptab_v, TBASE + NP - 1)
    plm2 = bc(ptab_v, TBASE + NP - 2)
    H0 = bc(htab_v, TBASE)
    H1 = bc(htab_v, TBASE + 1)
    Hm1 = bc(htab_v, TBASE + NP - 1)
    Hm2 = bc(htab_v, TBASE + NP - 2)
    slope0 = (H1 - H0) / (pl1v - pl0)
    first_H = jnp.where(pl0 > _c16(0.0, jnp.float32), H0 - slope0 * pl0, H0)
    slope1 = (Hm1 - Hm2) / (plm1 - plm2)
    tail_H = Hm1 + slope1 * (_c16(float(NFFT - 1), jnp.float32) - plm1)
    cond_tail = plm1 < _c16(float(NFFT - 1), jnp.float32)
    o_vec = jnp.where(cond_tail, _c16(1, jnp.int32), _c16(0, jnp.int32))
    last_p = jnp.maximum(plm1, _c16(float(NFFT - 1), jnp.float32))

    m0 = iota == _c16(0, jnp.int32)
    m01 = iota < _c16(2, jnp.int32)
    head_i = iota + _c16(TBASE - 2, jnp.int32)
    plsc.store_scatter(
        ptab_v, [head_i],
        jnp.where(m0, _c16(-1.0, jnp.float32), _c16(0.0, jnp.float32)),
        mask=m01)
    plsc.store_scatter(
        htab_v, [head_i],
        jnp.where(m0, _c16(0.0, jnp.float32), first_H),
        mask=m01)
    tail_i = _c16(TBASE + NP, jnp.int32)
    plsc.store_scatter(ptab_v, [tail_i], last_p, mask=m0)
    plsc.store_scatter(htab_v, [tail_i], tail_H, mask=m0)

    av = abg_v[pl.ds(0, L)]
    bv = abg_v[pl.ds(L, L)]
    gv = abg_v[pl.ds(2 * L, L)]
    base_out = wid * CHUNK
    g0_hi = o_vec + _c16(NP, jnp.int32)

    def count_le(idxf):
        pos = _c16(0, jnp.int32)
        bit = 1 << 13
        while bit:
            cand = pos + bit
            gidx = jnp.minimum(cand, _c16(NSRCH, jnp.int32)) \
                + _c16(TBASE - 1, jnp.int32)
            tv = plsc.load_gather(ptab_v, [gidx])
            take = jnp.logical_and(cand <= _c16(NSRCH, jnp.int32),
                                   tv <= idxf)
            pos = jnp.where(take, cand, pos)
            bit >>= 1
        return pos

    wlo = count_le((_c16(base_out, jnp.int32) - _c16(1, jnp.int32))
                   .astype(jnp.float32))
    whi = count_le(_c16(float(CHUNK - 1), jnp.float32)
                   + _c16(base_out, jnp.int32).astype(jnp.float32))
    wvec = whi - wlo
    w_s = lax.reduce_max(wvec, (0,))
    nsteps = jnp.int32(0)
    for k in range(14):
        nsteps = nsteps + jnp.where(w_s >= (1 << k), 1, 0).astype(jnp.int32)
    gbase = wlo + _c16(TBASE - 1, jnp.int32)

    OU = 8

    def out_body(i, carry):
        li_l, idxf_l, pos_l = [], [], []
        for u in range(OU):
            li = (i * OU + u) * L + iota
            li_l.append(li)
            idxf_l.append((base_out + li).astype(jnp.float32))
            pos_l.append(_c16(0, jnp.int32))

        def step(t, poss):
            bit = jnp.broadcast_to(
                lax.shift_left(jnp.int32(1), nsteps - 1 - t).astype(jnp.int32), (L,))
            poss = list(poss)
            for u in range(OU):
                cand = poss[u] + bit
                gidx = jnp.minimum(cand, wvec) + gbase
                tv = plsc.load_gather(ptab_v, [gidx])
                take = jnp.logical_and(cand <= wvec, tv <= idxf_l[u])
                poss[u] = jnp.where(take, cand, poss[u])
            return tuple(poss)

        pos_l = list(lax.fori_loop(0, nsteps, step, tuple(pos_l)))
        g0_l = [jnp.minimum(jnp.maximum(wlo + pos_l[u] + 1, o_vec), g0_hi)
                + _c16(TBASE - 2, jnp.int32) for u in range(OU)]
        x0_l = [plsc.load_gather(ptab_v, [g0_l[u]]) for u in range(OU)]
        x1_l = [plsc.load_gather(ptab_v, [g0_l[u] + 1]) for u in range(OU)]
        yb_l = [plsc.load_gather(htab_v, [g0_l[u]]) for u in range(OU)]
        ya_l = [plsc.load_gather(htab_v, [g0_l[u] + 1]) for u in range(OU)]
        for u in range(OU):
            denom = x1_l[u] - x0_l[u]
            safe = denom > _c16(0.0, jnp.float32)
            df = jnp.where(
                safe,
                (idxf_l[u] - x0_l[u])
                / jnp.where(safe, denom, _c16(1.0, jnp.float32)),
                _c16(0.0, jnp.float32))
            outv = av * ya_l[u] + bv * yb_l[u] + gv * df
            plsc.store_scatter(out_v, [li_l[u]], outv)
        return carry

    lax.fori_loop(0, CHUNK // L // OU, out_body, 0)

    pltpu.sync_copy(out_v, out_hbm.at[pl.ds(base_out, CHUNK)])


_estimator = functools.partial(
    pl.kernel,
    out_type=jax.ShapeDtypeStruct((NFFT,), jnp.float32),
    mesh=plsc.VectorSubcoreMesh(core_axis_name="c", subcore_axis_name="s",
                                num_cores=NC, num_subcores=NS),
    compiler_params=pltpu.CompilerParams(needs_layout_passes=False),
    scratch_types=[
        pltpu.VMEM((PCHUNK,), jnp.float32),
        pltpu.VMEM((PCHUNK,), jnp.int32),
        pltpu.VMEM((PCHUNK,), jnp.float32),
        pltpu.VMEM((PCHUNK,), jnp.float32),
        pltpu.VMEM((PCHUNK,), jnp.float32),
        pltpu.VMEM((PCHUNK,), jnp.float32),
        pltpu.VMEM((L,), jnp.int32),
        pltpu.VMEM((3 * L,), jnp.float32),
        pltpu.VMEM((TPAD,), jnp.float32),
        pltpu.VMEM((TPAD,), jnp.float32),
        pltpu.VMEM((CHUNK,), jnp.float32),
        pltpu.VMEM_SHARED((NP,), jnp.float32),
        pltpu.VMEM_SHARED((NP,), jnp.float32),
        pltpu.SemaphoreType.DMA,
        pltpu.SemaphoreType.DMA,
    ],
)(_sc_body)


def kernel(Y, Xp, pilot_pos, Nfft, estimation_weights, alpha, beta, gamma):
    del Nfft
    a = jnp.broadcast_to(jnp.reshape(jnp.asarray(alpha, jnp.float32), (1,)), (L,))
    b = jnp.broadcast_to(jnp.reshape(jnp.asarray(beta, jnp.float32), (1,)), (L,))
    g = jnp.broadcast_to(jnp.reshape(jnp.asarray(gamma, jnp.float32), (1,)), (L,))
    abg = jnp.concatenate([a, b, g])
    return _estimator(Y.astype(jnp.float32), Xp.astype(jnp.float32),
                      pilot_pos.astype(jnp.int32),
                      estimation_weights.astype(jnp.float32), abg)
